# baseline (speedup 1.0000x reference)
import jax
import jax.numpy as jnp
from jax import lax
from jax.experimental import pallas as pl

_B = 16384
_NLINES = 6
_BLK = _B
_GRID = _B // _BLK


def _encode_body(x_ref, idx_ref):
    x = x_ref[...].astype(jnp.float32)
    w = (jnp.int32(1) << lax.broadcasted_iota(
        jnp.int32, (_NLINES, _BLK), 0)).astype(jnp.float32)
    idx_ref[...] = jnp.sum(x * w, axis=0).astype(jnp.int32)


_encode = pl.pallas_call(
    _encode_body,
    grid=(_GRID,),
    in_specs=[pl.BlockSpec((_NLINES, _BLK), lambda i: (0, i))],
    out_specs=pl.BlockSpec((_BLK,), lambda i: (i,)),
    out_shape=jax.ShapeDtypeStruct((_B,), jnp.int32),
)


def kernel(lines, hex_table, line_table):
    hex_index = _encode(lines.T.astype(jnp.bfloat16))
    return (lines, hex_index, lines, jnp.zeros_like(lines))


# speedup vs baseline: 1.3240x; 1.3240x over previous
import jax
import jax.numpy as jnp
from jax import lax
from jax.experimental import pallas as pl

_B = 16384
_NLINES = 6
_BLK = _B
_GRID = _B // _BLK


def _encode_body(x_ref, idx_ref):
    x = x_ref[...]
    w = (jnp.int32(1) << lax.broadcasted_iota(
        jnp.int32, (_NLINES, _BLK), 0)).astype(jnp.float32)
    idx_ref[...] = jnp.sum(x * w, axis=0).astype(jnp.int32)


_encode = pl.pallas_call(
    _encode_body,
    grid=(_GRID,),
    in_specs=[pl.BlockSpec((_NLINES, _BLK), lambda i: (0, i))],
    out_specs=pl.BlockSpec((_BLK,), lambda i: (i,)),
    out_shape=jax.ShapeDtypeStruct((_B,), jnp.int32),
)


def kernel(lines, hex_table, line_table):
    hex_index = _encode(lines.T)
    return (lines, hex_index, lines, jnp.zeros_like(lines))
